# baseline (speedup 1.0000x reference)
import functools

import jax
import jax.numpy as jnp
from jax import lax
from jax.experimental import pallas as pl
from jax.experimental.pallas import tpu as pltpu
from jax.experimental.pallas import tpu_sc as plsc

D_MODEL = 64
LANE = 128
NUM_CORES = 2
NUM_SUBCORES = 16
NUM_WORKERS = NUM_CORES * NUM_SUBCORES


@functools.cache
def _build(seq, batch, vocab):
    assert batch == LANE * NUM_WORKERS and seq % 2 == 0
    n_bt = batch // LANE

    mesh = plsc.VectorSubcoreMesh(core_axis_name="c", subcore_axis_name="s")

    @functools.partial(
        pl.kernel,
        mesh=mesh,
        compiler_params=pltpu.CompilerParams(needs_layout_passes=False),
        out_type=jax.ShapeDtypeStruct((seq, 8, n_bt, 8, LANE), jnp.float32),
        scratch_types=[
            pltpu.VMEM((seq, LANE), jnp.int32),
            pltpu.VMEM((LANE, LANE), jnp.float32),
            pltpu.VMEM((LANE, LANE), jnp.float32),
            pltpu.VMEM((8, 8, LANE), jnp.float32),
            pltpu.VMEM((8, 8, LANE), jnp.float32),
            pltpu.SemaphoreType.DMA,
            pltpu.SemaphoreType.DMA,
            pltpu.SemaphoreType.DMA,
            pltpu.SemaphoreType.DMA,
        ],
    )
    def gather_kernel(tokt_hbm, table_hbm, out_hbm,
                      idxall, g0, g1, ob0, ob1, sg0, sg1, so0, so1):
        bt = lax.axis_index("s") * NUM_CORES + lax.axis_index("c")
        gs, obs, sgs, sos = (g0, g1), (ob0, ob1), (sg0, sg1), (so0, so1)

        def fire_gather(s, b):
            pltpu.async_copy(table_hbm.at[idxall.at[s]], gs[b], sgs[b])

        def wait_gather(b):
            pltpu.make_async_copy(table_hbm.at[idxall.at[0]], gs[b],
                                  sgs[b]).wait()

        def fire_out(s, b):
            for ad in range(8):
                pltpu.async_copy(obs[b].at[ad], out_hbm.at[s, ad, bt], sos[b])

        def wait_out(b):
            for ad in range(8):
                pltpu.make_async_copy(obs[b].at[0], out_hbm.at[0, 0, bt],
                                      sos[b]).wait()

        pltpu.sync_copy(tokt_hbm.at[:, pl.ds(bt * LANE, LANE)], idxall)

        iotas = [lax.iota(jnp.int32, 16) + 16 * cg for cg in range(8)]

        fire_gather(0, 0)

        @pl.loop(0, seq, step=2)
        def _(s0):
            for b in (0, 1):
                s = s0 + b
                nb = 1 - b
                wait_gather(b)

                @pl.when(s + 1 < seq)
                def _():
                    fire_gather(s + 1, nb)

                @pl.when(s >= 2)
                def _():
                    wait_out(b)

                for ad in range(8):
                    for r in range(8):
                        col = jnp.full((16,), 8 * ad + r, jnp.int32)
                        for cg in range(8):
                            obs[b][ad, r, pl.ds(16 * cg, 16)] = (
                                plsc.load_gather(gs[b], [iotas[cg], col]))
                fire_out(s, b)

        wait_out(0)
        wait_out(1)

    return gather_kernel


def kernel(token_ids, embedding_weight):
    batch, seq = token_ids.shape
    vocab, d = embedding_weight.shape
    tokt = token_ids.T.astype(jnp.int32)
    table128 = jnp.pad(embedding_weight, ((0, 0), (0, LANE - d)))
    out5 = _build(seq, batch, vocab)(tokt, table128)
    return out5.transpose(2, 4, 0, 1, 3).reshape(batch, seq, d)


# speedup vs baseline: 1.7819x; 1.7819x over previous
import functools

import jax
import jax.numpy as jnp
from jax import lax
from jax.experimental import pallas as pl
from jax.experimental.pallas import tpu as pltpu
from jax.experimental.pallas import tpu_sc as plsc

D_MODEL = 64
LANE = 128
NUM_CORES = 2
NUM_SUBCORES = 16
NUM_WORKERS = NUM_CORES * NUM_SUBCORES


@functools.cache
def _build(seq, batch, vocab):
    assert batch == LANE * NUM_WORKERS and seq % 2 == 0
    n_bt = batch // LANE

    mesh = plsc.VectorSubcoreMesh(core_axis_name="c", subcore_axis_name="s")

    @functools.partial(
        pl.kernel,
        mesh=mesh,
        compiler_params=pltpu.CompilerParams(needs_layout_passes=False),
        out_type=jax.ShapeDtypeStruct((seq, 8, n_bt, 8, LANE), jnp.float32),
        scratch_types=[
            pltpu.VMEM((seq, LANE), jnp.int32),
            pltpu.VMEM((LANE, LANE), jnp.float32),
            pltpu.VMEM((LANE, LANE), jnp.float32),
            pltpu.VMEM((D_MODEL, LANE), jnp.float32),
            pltpu.VMEM((D_MODEL, LANE), jnp.float32),
            pltpu.SemaphoreType.DMA,
            pltpu.SemaphoreType.DMA,
            pltpu.SemaphoreType.DMA,
            pltpu.SemaphoreType.DMA,
        ],
    )
    def gather_kernel(tokt_hbm, table_hbm, out_hbm,
                      idxall, g0, g1, ob0, ob1, sg0, sg1, so0, so1):
        bt = lax.axis_index("s") * NUM_CORES + lax.axis_index("c")
        gs, obs, sgs, sos = (g0, g1), (ob0, ob1), (sg0, sg1), (so0, so1)

        def fire_gather(s, b):
            pltpu.async_copy(table_hbm.at[idxall.at[s]], gs[b], sgs[b])

        def wait_gather(b):
            pltpu.make_async_copy(table_hbm.at[idxall.at[0]], gs[b],
                                  sgs[b]).wait()

        def fire_out(s, b):
            for ad in range(8):
                pltpu.async_copy(obs[b].at[pl.ds(8 * ad, 8)],
                                 out_hbm.at[s, ad, bt], sos[b])

        def wait_out(b):
            for ad in range(8):
                pltpu.make_async_copy(obs[b].at[pl.ds(0, 8)],
                                      out_hbm.at[0, 0, bt], sos[b]).wait()

        pltpu.sync_copy(tokt_hbm.at[:, pl.ds(bt * LANE, LANE)], idxall)

        iota = lax.iota(jnp.int32, 16)
        rows_vs = [iota + 16 * rb for rb in range(8)]
        perms = [(iota + d) % 16 for d in range(16)]

        fire_gather(0, 0)

        @pl.loop(0, seq, step=2)
        def _(s0):
            for b in (0, 1):
                s = s0 + b
                nb = 1 - b
                wait_gather(b)

                @pl.when(s + 1 < seq)
                def _():
                    fire_gather(s + 1, nb)

                @pl.when(s >= 2)
                def _():
                    wait_out(b)

                @pl.loop(0, 8)
                def _(rb):
                    rows_v = iota + 16 * rb
                    for cb in range(D_MODEL // 16):
                        for d in range(16):
                            cols_v = perms[d] + 16 * cb
                            v = plsc.load_gather(gs[b], [rows_v, cols_v])
                            plsc.store_scatter(obs[b], [cols_v, rows_v], v)
                fire_out(s, b)

        wait_out(0)
        wait_out(1)

    return gather_kernel


def kernel(token_ids, embedding_weight):
    batch, seq = token_ids.shape
    vocab, d = embedding_weight.shape
    tokt = token_ids.T.astype(jnp.int32)
    table128 = jnp.pad(embedding_weight, ((0, 0), (0, LANE - d)))
    out5 = _build(seq, batch, vocab)(tokt, table128)
    return out5.transpose(2, 4, 0, 1, 3).reshape(batch, seq, d)


# speedup vs baseline: 2.2855x; 1.2826x over previous
import functools

import jax
import jax.numpy as jnp
from jax import lax
from jax.experimental import pallas as pl
from jax.experimental.pallas import tpu as pltpu
from jax.experimental.pallas import tpu_sc as plsc

D_MODEL = 64
LANE = 128
NUM_CORES = 2
NUM_SUBCORES = 16
NUM_WORKERS = NUM_CORES * NUM_SUBCORES


@functools.cache
def _build(seq, batch, vocab):
    assert batch == LANE * NUM_WORKERS and seq % 2 == 0
    n_bt = batch // LANE

    mesh = plsc.VectorSubcoreMesh(core_axis_name="c", subcore_axis_name="s")

    @functools.partial(
        pl.kernel,
        mesh=mesh,
        compiler_params=pltpu.CompilerParams(needs_layout_passes=False),
        out_type=jax.ShapeDtypeStruct((seq, 8, n_bt, 8, LANE), jnp.float32),
        scratch_types=[
            pltpu.VMEM((seq, LANE), jnp.int32),
            pltpu.VMEM((LANE, LANE), jnp.float32),
            pltpu.VMEM((LANE, LANE), jnp.float32),
            pltpu.VMEM((D_MODEL, LANE), jnp.float32),
            pltpu.VMEM((D_MODEL, LANE), jnp.float32),
            pltpu.SemaphoreType.DMA,
            pltpu.SemaphoreType.DMA,
            pltpu.SemaphoreType.DMA,
            pltpu.SemaphoreType.DMA,
        ],
    )
    def gather_kernel(tokt_hbm, table_hbm, out_hbm,
                      idxall, g0, g1, ob0, ob1, sg0, sg1, so0, so1):
        bt = lax.axis_index("s") * NUM_CORES + lax.axis_index("c")
        gs, obs, sgs, sos = (g0, g1), (ob0, ob1), (sg0, sg1), (so0, so1)

        def fire_gather(s, b):
            pltpu.async_copy(table_hbm.at[idxall.at[s]], gs[b], sgs[b])

        def wait_gather(b):
            pltpu.make_async_copy(table_hbm.at[idxall.at[0]], gs[b],
                                  sgs[b]).wait()

        def fire_out(s, b):
            for ad in range(8):
                pltpu.async_copy(obs[b].at[pl.ds(8 * ad, 8)],
                                 out_hbm.at[s, ad, bt], sos[b])

        def wait_out(b):
            for ad in range(8):
                pltpu.make_async_copy(obs[b].at[pl.ds(0, 8)],
                                      out_hbm.at[0, 0, bt], sos[b]).wait()

        pltpu.sync_copy(tokt_hbm.at[:, pl.ds(bt * LANE, LANE)], idxall)

        iota = lax.iota(jnp.int32, 16)
        rows_vs = [iota + 16 * rb for rb in range(8)]
        perms = [(iota + d) % 16 for d in range(16)]

        fire_gather(0, 0)

        @pl.loop(0, seq, step=2)
        def _(s0):
            for b in (0, 1):
                s = s0 + b
                nb = 1 - b
                wait_gather(b)

                @pl.when(s + 1 < seq)
                def _():
                    fire_gather(s + 1, nb)

                @pl.when(s >= 2)
                def _():
                    wait_out(b)

                @plsc.parallel_loop(0, 512, unroll=8)
                def _(i):
                    rows_v = iota + ((i >> 2) & 0x70)
                    cols_v = ((iota + (i & 15)) & 15) | (i & 0x30)
                    v = plsc.load_gather(gs[b], [rows_v, cols_v])
                    plsc.store_scatter(obs[b], [cols_v, rows_v], v)
                fire_out(s, b)

        wait_out(0)
        wait_out(1)

    return gather_kernel


def kernel(token_ids, embedding_weight):
    batch, seq = token_ids.shape
    vocab, d = embedding_weight.shape
    tokt = token_ids.T.astype(jnp.int32)
    table128 = jnp.pad(embedding_weight, ((0, 0), (0, LANE - d)))
    out5 = _build(seq, batch, vocab)(tokt, table128)
    return out5.transpose(2, 4, 0, 1, 3).reshape(batch, seq, d)
